# baseline (speedup 1.0000x reference)
import jax
import jax.numpy as jnp
from jax import lax
from jax.experimental import pallas as pl
from jax.experimental.pallas import tpu as pltpu
from jax.experimental.pallas import tpu_sc as plsc

NC = 2
NS = 16
L = 16
NW = NC * NS
T = 409600
NUM_ENT = 1000000
NB = 8192
D = 64
EPW = T // NS
WIN = 128
NWIN = EPW // WIN
RPT = NB // NS

_MESH = plsc.VectorSubcoreMesh(
    core_axis_name="c", subcore_axis_name="s", num_cores=NC, num_subcores=NS
)


def _sc_body(ids_hbm, off_hbm, ents_hbm, etab_hbm, rtab_hbm,
             acc_out, emb_out,
             acc_sh, offs_v, seg_v, ids_v, erows_v, nids_v):
    cid = lax.axis_index("c")
    sid = lax.axis_index("s")
    wid = cid * NS + sid
    lo = sid * EPW

    pltpu.sync_copy(off_hbm, offs_v)
    pltpu.sync_copy(ids_hbm.at[cid, sid], ids_v)
    pltpu.sync_copy(ents_hbm.at[cid, sid], nids_v)

    zi = jnp.zeros((L,), jnp.int32)
    zf = jnp.zeros((L,), jnp.float32)
    CPR = WIN // L

    @pl.loop(0, EPW // L)
    def _(i):
        seg_v[i // CPR, pl.ds((i % CPR) * L, L)] = zi

    @pl.loop(0, WIN * D // L)
    def _(i):
        erows_v[i // (D // L), pl.ds((i % (D // L)) * L, L)] = zf

    for j in range(RPT // WIN):
        r0 = sid * RPT + j * WIN
        pltpu.sync_copy(erows_v, acc_sh.at[pl.ds(r0, WIN)])

    ones = jnp.ones((L,), jnp.int32)

    def _scatter_offsets(k, carry):
        v = offs_v[pl.ds(k * L, L)]
        rel = v - lo
        m_in = (rel >= 0) & (rel < EPW)
        plsc.addupdate_scatter(seg_v, [rel // WIN, rel % WIN], ones, mask=m_in)
        return carry + jnp.where(v < lo, 1, 0)

    lt_lanes = lax.fori_loop(0, NB // L, _scatter_offsets,
                             jnp.zeros((L,), jnp.int32))
    c0 = jnp.sum(lt_lanes)

    def _cumsum_row(j, carry):
        v = seg_v[j // CPR, pl.ds((j % CPR) * L, L)]
        c = plsc.cumsum(v) + carry
        seg_v[j // CPR, pl.ds((j % CPR) * L, L)] = c
        return jnp.max(c)

    lax.fori_loop(0, EPW // L, _cumsum_row, c0 - 1)

    plsc.subcore_barrier()

    @pl.when(cid == 0)
    def _():
        @pl.loop(0, NWIN)
        def _(w):
            pltpu.sync_copy(etab_hbm.at[ids_v.at[w]], erows_v)
            pltpu.sync_copy(erows_v, acc_sh.at[seg_v.at[w]], add=True)

    @pl.when(cid == 1)
    def _():
        @pl.loop(0, NWIN)
        def _(w):
            pltpu.sync_copy(rtab_hbm.at[ids_v.at[w]], erows_v)
            pltpu.sync_copy(erows_v, acc_sh.at[seg_v.at[w]], add=True)

    for j in range(2):
        pltpu.sync_copy(etab_hbm.at[nids_v.at[j]], erows_v)
        pltpu.sync_copy(erows_v, emb_out.at[pl.ds(wid * 2 * WIN + j * WIN, WIN)])

    plsc.subcore_barrier()
    for j in range(RPT // WIN):
        r0 = sid * RPT + j * WIN
        pltpu.sync_copy(acc_sh.at[pl.ds(r0, WIN)], acc_out.at[cid, pl.ds(r0, WIN)])


_sc_call = pl.kernel(
    _sc_body,
    out_type=(
        jax.ShapeDtypeStruct((NC, NB, D), jnp.float32),
        jax.ShapeDtypeStruct((NB, D), jnp.float32),
    ),
    mesh=_MESH,
    compiler_params=pltpu.CompilerParams(
        needs_layout_passes=False, use_tc_tiling_on_sc=False
    ),
    scratch_types=[
        pltpu.VMEM_SHARED((NB, D), jnp.float32),
        pltpu.VMEM((NB,), jnp.int32),
        pltpu.VMEM((NWIN, WIN), jnp.int32),
        pltpu.VMEM((NWIN, WIN), jnp.int32),
        pltpu.VMEM((WIN, D), jnp.float32),
        pltpu.VMEM((2, WIN), jnp.int32),
    ],
)


def _tcT_body(xt, eye, out):
    out[...] = lax.dot_general(xt[...], eye[...], (((0,), (0,)), ((), ())),
                               precision=lax.Precision.HIGHEST,
                               preferred_element_type=jnp.float32)


def _tc1_body(acc_e, acc_r, emb, olo, ohi, wt, wne, wnr, bt, bn, node_out):
    cnt = (ohi[...] - olo[...]).astype(jnp.float32)
    inv = 1.0 / jnp.maximum(cnt, 1.0)
    bag_e = acc_e[...] * inv
    bag_r = acc_r[...] * inv
    dn = (((1,), (1,)), ((), ()))
    ent_trans = lax.dot_general(emb[...], wt[...], dn,
                                precision=lax.Precision.HIGHEST,
                                preferred_element_type=jnp.float32)
    neigh = (lax.dot_general(bag_e, wne[...], dn,
                             precision=lax.Precision.HIGHEST,
                             preferred_element_type=jnp.float32)
             + lax.dot_general(bag_r, wnr[...], dn,
                               precision=lax.Precision.HIGHEST,
                               preferred_element_type=jnp.float32))
    node_out[...] = jnp.maximum(ent_trans + neigh + bt[...] + bn[...], 0.0)


def _tc2_body(node2, wr, br, pair_out):
    dn = (((1,), (1,)), ((), ()))
    pair = lax.dot_general(node2[...], wr[...], dn,
                           precision=lax.Precision.HIGHEST,
                           preferred_element_type=jnp.float32)
    pair_out[...] = jnp.maximum(pair + br[...], 0.0)


def kernel(entities, neighbor_entities, neighbor_relations, offsets,
           entity_table, relation_table, W_t, b_t, W_n, b_n, W_r, b_r):
    entities = entities.astype(jnp.int32).reshape(NC, NS, 2, WIN)
    ne = neighbor_entities.astype(jnp.int32).reshape(NS, NWIN, WIN)
    nr = neighbor_relations.astype(jnp.int32).reshape(NS, NWIN, WIN)
    ids = jnp.stack([ne, nr])
    offsets = offsets.astype(jnp.int32)
    BCT = 2048
    etabT = entity_table.T
    eye = jnp.eye(D, dtype=jnp.float32)
    entity_table = pl.pallas_call(
        _tcT_body,
        grid=(pl.cdiv(NUM_ENT, BCT),),
        in_specs=[
            pl.BlockSpec((D, BCT), lambda i: (0, i)),
            pl.BlockSpec((D, D), lambda i: (0, 0)),
        ],
        out_specs=pl.BlockSpec((BCT, D), lambda i: (i, 0)),
        out_shape=jax.ShapeDtypeStruct((NUM_ENT, D), jnp.float32),
    )(etabT, eye)

    acc, emb = _sc_call(ids, offsets, entities, entity_table, relation_table)
    acc_e = acc[0]
    acc_r = acc[1]

    olo = offsets.reshape(NB, 1)
    ohi = jnp.concatenate([offsets[1:], jnp.full((1,), T, jnp.int32)]).reshape(NB, 1)
    wne = W_n[:, :D]
    wnr = W_n[:, D:]

    BR1 = 1024
    node = pl.pallas_call(
        _tc1_body,
        grid=(NB // BR1,),
        in_specs=[
            pl.BlockSpec((BR1, D), lambda i: (i, 0)),
            pl.BlockSpec((BR1, D), lambda i: (i, 0)),
            pl.BlockSpec((BR1, D), lambda i: (i, 0)),
            pl.BlockSpec((BR1, 1), lambda i: (i, 0)),
            pl.BlockSpec((BR1, 1), lambda i: (i, 0)),
            pl.BlockSpec((2 * D, D), lambda i: (0, 0)),
            pl.BlockSpec((2 * D, D), lambda i: (0, 0)),
            pl.BlockSpec((2 * D, D), lambda i: (0, 0)),
            pl.BlockSpec((1, 2 * D), lambda i: (0, 0)),
            pl.BlockSpec((1, 2 * D), lambda i: (0, 0)),
        ],
        out_specs=pl.BlockSpec((BR1, 2 * D), lambda i: (i, 0)),
        out_shape=jax.ShapeDtypeStruct((NB, 2 * D), jnp.float32),
    )(acc_e, acc_r, emb, olo, ohi, W_t, wne, wnr,
      b_t.reshape(1, 2 * D), b_n.reshape(1, 2 * D))

    node2 = node.reshape(NB // 2, 4 * D)
    BR2 = 1024
    pair = pl.pallas_call(
        _tc2_body,
        grid=(NB // 2 // BR2,),
        in_specs=[
            pl.BlockSpec((BR2, 4 * D), lambda i: (i, 0)),
            pl.BlockSpec((2 * D, 4 * D), lambda i: (0, 0)),
            pl.BlockSpec((1, 2 * D), lambda i: (0, 0)),
        ],
        out_specs=pl.BlockSpec((BR2, 2 * D), lambda i: (i, 0)),
        out_shape=jax.ShapeDtypeStruct((NB // 2, 2 * D), jnp.float32),
    )(node2, W_r, b_r.reshape(1, 2 * D))
    return pair


# speedup vs baseline: 1.1026x; 1.1026x over previous
import jax
import jax.numpy as jnp
from jax import lax
from jax.experimental import pallas as pl
from jax.experimental.pallas import tpu as pltpu
from jax.experimental.pallas import tpu_sc as plsc

NC = 2
NS = 16
L = 16
NW = NC * NS
T = 409600
NUM_ENT = 1000000
NB = 8192
D = 64
EPW = T // NS
WIN = 128
NWIN = EPW // WIN
RPT = NB // NS

_MESH = plsc.VectorSubcoreMesh(
    core_axis_name="c", subcore_axis_name="s", num_cores=NC, num_subcores=NS
)


def _sc_body(ids_hbm, off_hbm, ents_hbm, etab_hbm, rtab_hbm,
             acc_out, emb_out,
             acc_sh, offs_v, seg_v, ids_v, erows_v, nids_v):
    cid = lax.axis_index("c")
    sid = lax.axis_index("s")
    wid = cid * NS + sid
    lo = sid * EPW

    pltpu.sync_copy(off_hbm, offs_v)
    pltpu.sync_copy(ids_hbm.at[cid, sid], ids_v)
    pltpu.sync_copy(ents_hbm.at[cid, sid], nids_v)

    zi = jnp.zeros((L,), jnp.int32)
    zf = jnp.zeros((L,), jnp.float32)
    CPR = WIN // L

    @pl.loop(0, EPW // L)
    def _(i):
        seg_v[i // CPR, pl.ds((i % CPR) * L, L)] = zi

    @pl.loop(0, WIN * D // L)
    def _(i):
        erows_v[i // (D // L), pl.ds((i % (D // L)) * L, L)] = zf

    for j in range(RPT // WIN):
        r0 = sid * RPT + j * WIN
        pltpu.sync_copy(erows_v, acc_sh.at[pl.ds(r0, WIN)])

    ones = jnp.ones((L,), jnp.int32)

    def _scatter_offsets(k, carry):
        v = offs_v[pl.ds(k * L, L)]
        rel = v - lo
        m_in = (rel >= 0) & (rel < EPW)
        plsc.addupdate_scatter(seg_v, [rel // WIN, rel % WIN], ones, mask=m_in)
        return carry + jnp.where(v < lo, 1, 0)

    lt_lanes = lax.fori_loop(0, NB // L, _scatter_offsets,
                             jnp.zeros((L,), jnp.int32))
    c0 = jnp.sum(lt_lanes)

    def _cumsum_row(j, carry):
        v = seg_v[j // CPR, pl.ds((j % CPR) * L, L)]
        c = plsc.cumsum(v) + carry
        seg_v[j // CPR, pl.ds((j % CPR) * L, L)] = c
        return jnp.max(c)

    lax.fori_loop(0, EPW // L, _cumsum_row, c0 - 1)

    plsc.subcore_barrier()

    @pl.when(cid == 0)
    def _():
        @pl.loop(0, NWIN)
        def _(w):
            pltpu.sync_copy(etab_hbm.at[ids_v.at[w]], erows_v)
            pltpu.sync_copy(erows_v, acc_sh.at[seg_v.at[w]], add=True)

    @pl.when(cid == 1)
    def _():
        @pl.loop(0, NWIN)
        def _(w):
            pltpu.sync_copy(rtab_hbm.at[ids_v.at[w]], erows_v)
            pltpu.sync_copy(erows_v, acc_sh.at[seg_v.at[w]], add=True)

    for j in range(2):
        pltpu.sync_copy(etab_hbm.at[nids_v.at[j]], erows_v)
        pltpu.sync_copy(erows_v, emb_out.at[pl.ds(wid * 2 * WIN + j * WIN, WIN)])

    plsc.subcore_barrier()
    for j in range(RPT // WIN):
        r0 = sid * RPT + j * WIN
        pltpu.sync_copy(acc_sh.at[pl.ds(r0, WIN)], acc_out.at[cid, pl.ds(r0, WIN)])


_sc_call = pl.kernel(
    _sc_body,
    out_type=(
        jax.ShapeDtypeStruct((NC, NB, D), jnp.float32),
        jax.ShapeDtypeStruct((NB, D), jnp.float32),
    ),
    mesh=_MESH,
    compiler_params=pltpu.CompilerParams(
        needs_layout_passes=False, use_tc_tiling_on_sc=False
    ),
    scratch_types=[
        pltpu.VMEM_SHARED((NB, D), jnp.float32),
        pltpu.VMEM((NB,), jnp.int32),
        pltpu.VMEM((NWIN, WIN), jnp.int32),
        pltpu.VMEM((NWIN, WIN), jnp.int32),
        pltpu.VMEM((WIN, D), jnp.float32),
        pltpu.VMEM((2, WIN), jnp.int32),
    ],
)


def _tcT_body(xt, eye, out):
    out[...] = lax.dot_general(xt[...], eye[...], (((0,), (0,)), ((), ())),
                               preferred_element_type=jnp.float32)


def _tc1_body(acc_e, acc_r, emb, olo, ohi, wt, wne, wnr, bt, bn, node_out):
    cnt = (ohi[...] - olo[...]).astype(jnp.float32)
    inv = 1.0 / jnp.maximum(cnt, 1.0)
    bag_e = acc_e[...] * inv
    bag_r = acc_r[...] * inv
    dn = (((1,), (1,)), ((), ()))
    ent_trans = lax.dot_general(emb[...], wt[...], dn,
                                precision=lax.Precision.HIGHEST,
                                preferred_element_type=jnp.float32)
    neigh = (lax.dot_general(bag_e, wne[...], dn,
                             precision=lax.Precision.HIGHEST,
                             preferred_element_type=jnp.float32)
             + lax.dot_general(bag_r, wnr[...], dn,
                               precision=lax.Precision.HIGHEST,
                               preferred_element_type=jnp.float32))
    node_out[...] = jnp.maximum(ent_trans + neigh + bt[...] + bn[...], 0.0)


def _tc2_body(node2, wr, br, pair_out):
    dn = (((1,), (1,)), ((), ()))
    pair = lax.dot_general(node2[...], wr[...], dn,
                           precision=lax.Precision.HIGHEST,
                           preferred_element_type=jnp.float32)
    pair_out[...] = jnp.maximum(pair + br[...], 0.0)


def kernel(entities, neighbor_entities, neighbor_relations, offsets,
           entity_table, relation_table, W_t, b_t, W_n, b_n, W_r, b_r):
    entities = entities.astype(jnp.int32).reshape(NC, NS, 2, WIN)
    ne = neighbor_entities.astype(jnp.int32).reshape(NS, NWIN, WIN)
    nr = neighbor_relations.astype(jnp.int32).reshape(NS, NWIN, WIN)
    ids = jnp.stack([ne, nr])
    offsets = offsets.astype(jnp.int32)
    BCT = 2048
    etabT = entity_table.T
    eye = jnp.eye(D, dtype=jnp.float32)
    entity_table = pl.pallas_call(
        _tcT_body,
        grid=(pl.cdiv(NUM_ENT, BCT),),
        in_specs=[
            pl.BlockSpec((D, BCT), lambda i: (0, i)),
            pl.BlockSpec((D, D), lambda i: (0, 0)),
        ],
        out_specs=pl.BlockSpec((BCT, D), lambda i: (i, 0)),
        out_shape=jax.ShapeDtypeStruct((NUM_ENT, D), jnp.float32),
    )(etabT, eye)

    acc, emb = _sc_call(ids, offsets, entities, entity_table, relation_table)
    acc_e = acc[0]
    acc_r = acc[1]

    olo = offsets.reshape(NB, 1)
    ohi = jnp.concatenate([offsets[1:], jnp.full((1,), T, jnp.int32)]).reshape(NB, 1)
    wne = W_n[:, :D]
    wnr = W_n[:, D:]

    BR1 = 1024
    node = pl.pallas_call(
        _tc1_body,
        grid=(NB // BR1,),
        in_specs=[
            pl.BlockSpec((BR1, D), lambda i: (i, 0)),
            pl.BlockSpec((BR1, D), lambda i: (i, 0)),
            pl.BlockSpec((BR1, D), lambda i: (i, 0)),
            pl.BlockSpec((BR1, 1), lambda i: (i, 0)),
            pl.BlockSpec((BR1, 1), lambda i: (i, 0)),
            pl.BlockSpec((2 * D, D), lambda i: (0, 0)),
            pl.BlockSpec((2 * D, D), lambda i: (0, 0)),
            pl.BlockSpec((2 * D, D), lambda i: (0, 0)),
            pl.BlockSpec((1, 2 * D), lambda i: (0, 0)),
            pl.BlockSpec((1, 2 * D), lambda i: (0, 0)),
        ],
        out_specs=pl.BlockSpec((BR1, 2 * D), lambda i: (i, 0)),
        out_shape=jax.ShapeDtypeStruct((NB, 2 * D), jnp.float32),
    )(acc_e, acc_r, emb, olo, ohi, W_t, wne, wnr,
      b_t.reshape(1, 2 * D), b_n.reshape(1, 2 * D))

    node2 = node.reshape(NB // 2, 4 * D)
    BR2 = 1024
    pair = pl.pallas_call(
        _tc2_body,
        grid=(NB // 2 // BR2,),
        in_specs=[
            pl.BlockSpec((BR2, 4 * D), lambda i: (i, 0)),
            pl.BlockSpec((2 * D, 4 * D), lambda i: (0, 0)),
            pl.BlockSpec((1, 2 * D), lambda i: (0, 0)),
        ],
        out_specs=pl.BlockSpec((BR2, 2 * D), lambda i: (i, 0)),
        out_shape=jax.ShapeDtypeStruct((NB // 2, 2 * D), jnp.float32),
    )(node2, W_r, b_r.reshape(1, 2 * D))
    return pair


# speedup vs baseline: 1.6421x; 1.4893x over previous
import jax
import jax.numpy as jnp
from jax import lax
from jax.experimental import pallas as pl
from jax.experimental.pallas import tpu as pltpu
from jax.experimental.pallas import tpu_sc as plsc

NC = 2
NS = 16
L = 16
NW = NC * NS
T = 409600
NUM_ENT = 1000000
NB = 8192
D = 64
EPW = T // NS
WIN = 128
NWIN = EPW // WIN
RPT = NB // NS
NSLOT = 4

_MESH = plsc.VectorSubcoreMesh(
    core_axis_name="c", subcore_axis_name="s", num_cores=NC, num_subcores=NS
)


def _sc_body(ids_hbm, off_hbm, ents_hbm, etab_hbm, rtab_hbm,
             acc_out, emb_out,
             acc_sh, offs_v, seg_v, ids_v, erows_v, nids_v, gsem, ssem):
    cid = lax.axis_index("c")
    sid = lax.axis_index("s")
    wid = cid * NS + sid
    lo = sid * EPW

    pltpu.sync_copy(off_hbm, offs_v)
    pltpu.sync_copy(ids_hbm.at[cid, sid], ids_v)
    pltpu.sync_copy(ents_hbm.at[cid, sid], nids_v)

    zi = jnp.zeros((L,), jnp.int32)
    zf = jnp.zeros((L,), jnp.float32)
    CPR = WIN // L

    @pl.loop(0, EPW // L)
    def _(i):
        seg_v[i // CPR, pl.ds((i % CPR) * L, L)] = zi

    @pl.loop(0, WIN * D // L)
    def _(i):
        erows_v[0, i // (D // L), pl.ds((i % (D // L)) * L, L)] = zf

    for j in range(RPT // WIN):
        r0 = sid * RPT + j * WIN
        pltpu.sync_copy(erows_v.at[0], acc_sh.at[pl.ds(r0, WIN)])

    ones = jnp.ones((L,), jnp.int32)

    def _scatter_offsets(k, carry):
        v = offs_v[pl.ds(k * L, L)]
        rel = v - lo
        m_in = (rel >= 0) & (rel < EPW)
        plsc.addupdate_scatter(seg_v, [rel // WIN, rel % WIN], ones, mask=m_in)
        return carry + jnp.where(v < lo, 1, 0)

    lt_lanes = lax.fori_loop(0, NB // L, _scatter_offsets,
                             jnp.zeros((L,), jnp.int32))
    c0 = jnp.sum(lt_lanes)

    def _cumsum_row(j, carry):
        v = seg_v[j // CPR, pl.ds((j % CPR) * L, L)]
        c = plsc.cumsum(v) + carry
        seg_v[j // CPR, pl.ds((j % CPR) * L, L)] = c
        return jnp.max(c)

    lax.fori_loop(0, EPW // L, _cumsum_row, c0 - 1)

    plsc.subcore_barrier()

    def _edge_loop(tab_hbm):
        for b in range(NSLOT):
            pltpu.async_copy(tab_hbm.at[ids_v.at[b]], erows_v.at[b],
                             gsem.at[b])

        @pl.loop(0, NWIN // NSLOT)
        def _(g):
            for b in range(NSLOT):
                w = g * NSLOT + b
                pltpu.make_async_copy(tab_hbm.at[ids_v.at[w]],
                                      erows_v.at[b], gsem.at[b]).wait()
                pltpu.async_copy(erows_v.at[b], acc_sh.at[seg_v.at[w]],
                                 ssem.at[b], add=True)
                pltpu.make_async_copy(erows_v.at[b],
                                      acc_sh.at[seg_v.at[w]],
                                      ssem.at[b]).wait()

                @pl.when(g < NWIN // NSLOT - 1)
                def _():
                    pltpu.async_copy(tab_hbm.at[ids_v.at[w + NSLOT]],
                                     erows_v.at[b], gsem.at[b])

    @pl.when(cid == 0)
    def _():
        _edge_loop(etab_hbm)

    @pl.when(cid == 1)
    def _():
        _edge_loop(rtab_hbm)

    for j in range(2):
        pltpu.sync_copy(etab_hbm.at[nids_v.at[j]], erows_v.at[0])
        pltpu.sync_copy(erows_v.at[0],
                        emb_out.at[pl.ds(wid * 2 * WIN + j * WIN, WIN)])

    plsc.subcore_barrier()
    for j in range(RPT // WIN):
        r0 = sid * RPT + j * WIN
        pltpu.sync_copy(acc_sh.at[pl.ds(r0, WIN)], acc_out.at[cid, pl.ds(r0, WIN)])


_sc_call = pl.kernel(
    _sc_body,
    out_type=(
        jax.ShapeDtypeStruct((NC, NB, D), jnp.float32),
        jax.ShapeDtypeStruct((NB, D), jnp.float32),
    ),
    mesh=_MESH,
    compiler_params=pltpu.CompilerParams(
        needs_layout_passes=False, use_tc_tiling_on_sc=False
    ),
    scratch_types=[
        pltpu.VMEM_SHARED((NB, D), jnp.float32),
        pltpu.VMEM((NB,), jnp.int32),
        pltpu.VMEM((NWIN, WIN), jnp.int32),
        pltpu.VMEM((NWIN, WIN), jnp.int32),
        pltpu.VMEM((NSLOT, WIN, D), jnp.float32),
        pltpu.VMEM((2, WIN), jnp.int32),
        pltpu.SemaphoreType.DMA((NSLOT,)),
        pltpu.SemaphoreType.DMA((NSLOT,)),
    ],
)


def _tc1_body(acc_e, acc_r, emb, olo, ohi, wt, wne, wnr, bt, bn, node_out):
    cnt = (ohi[...] - olo[...]).astype(jnp.float32)
    inv = 1.0 / jnp.maximum(cnt, 1.0)
    bag_e = acc_e[...] * inv
    bag_r = acc_r[...] * inv
    dn = (((1,), (1,)), ((), ()))
    ent_trans = lax.dot_general(emb[...], wt[...], dn,
                                precision=lax.Precision.HIGHEST,
                                preferred_element_type=jnp.float32)
    neigh = (lax.dot_general(bag_e, wne[...], dn,
                             precision=lax.Precision.HIGHEST,
                             preferred_element_type=jnp.float32)
             + lax.dot_general(bag_r, wnr[...], dn,
                               precision=lax.Precision.HIGHEST,
                               preferred_element_type=jnp.float32))
    node_out[...] = jnp.maximum(ent_trans + neigh + bt[...] + bn[...], 0.0)


def _tc2_body(node2, wr, br, pair_out):
    dn = (((1,), (1,)), ((), ()))
    pair = lax.dot_general(node2[...], wr[...], dn,
                           precision=lax.Precision.HIGHEST,
                           preferred_element_type=jnp.float32)
    pair_out[...] = jnp.maximum(pair + br[...], 0.0)


def kernel(entities, neighbor_entities, neighbor_relations, offsets,
           entity_table, relation_table, W_t, b_t, W_n, b_n, W_r, b_r):
    entities = entities.astype(jnp.int32).reshape(NC, NS, 2, WIN)
    ne = neighbor_entities.astype(jnp.int32).reshape(NS, NWIN, WIN)
    nr = neighbor_relations.astype(jnp.int32).reshape(NS, NWIN, WIN)
    ids = jnp.stack([ne, nr])
    offsets = offsets.astype(jnp.int32)

    acc, emb = _sc_call(ids, offsets, entities, entity_table, relation_table)
    acc_e = acc[0]
    acc_r = acc[1]

    olo = offsets.reshape(NB, 1)
    ohi = jnp.concatenate([offsets[1:], jnp.full((1,), T, jnp.int32)]).reshape(NB, 1)
    wne = W_n[:, :D]
    wnr = W_n[:, D:]

    BR1 = 1024
    node = pl.pallas_call(
        _tc1_body,
        grid=(NB // BR1,),
        in_specs=[
            pl.BlockSpec((BR1, D), lambda i: (i, 0)),
            pl.BlockSpec((BR1, D), lambda i: (i, 0)),
            pl.BlockSpec((BR1, D), lambda i: (i, 0)),
            pl.BlockSpec((BR1, 1), lambda i: (i, 0)),
            pl.BlockSpec((BR1, 1), lambda i: (i, 0)),
            pl.BlockSpec((2 * D, D), lambda i: (0, 0)),
            pl.BlockSpec((2 * D, D), lambda i: (0, 0)),
            pl.BlockSpec((2 * D, D), lambda i: (0, 0)),
            pl.BlockSpec((1, 2 * D), lambda i: (0, 0)),
            pl.BlockSpec((1, 2 * D), lambda i: (0, 0)),
        ],
        out_specs=pl.BlockSpec((BR1, 2 * D), lambda i: (i, 0)),
        out_shape=jax.ShapeDtypeStruct((NB, 2 * D), jnp.float32),
    )(acc_e, acc_r, emb, olo, ohi, W_t, wne, wnr,
      b_t.reshape(1, 2 * D), b_n.reshape(1, 2 * D))

    node2 = node.reshape(NB // 2, 4 * D)
    BR2 = 1024
    pair = pl.pallas_call(
        _tc2_body,
        grid=(NB // 2 // BR2,),
        in_specs=[
            pl.BlockSpec((BR2, 4 * D), lambda i: (i, 0)),
            pl.BlockSpec((2 * D, 4 * D), lambda i: (0, 0)),
            pl.BlockSpec((1, 2 * D), lambda i: (0, 0)),
        ],
        out_specs=pl.BlockSpec((BR2, 2 * D), lambda i: (i, 0)),
        out_shape=jax.ShapeDtypeStruct((NB // 2, 2 * D), jnp.float32),
    )(node2, W_r, b_r.reshape(1, 2 * D))
    return pair


# speedup vs baseline: 1.9402x; 1.1815x over previous
import jax
import jax.numpy as jnp
from jax import lax
from jax.experimental import pallas as pl
from jax.experimental.pallas import tpu as pltpu
from jax.experimental.pallas import tpu_sc as plsc

NC = 2
NS = 16
L = 16
NW = NC * NS
T = 409600
NUM_ENT = 1000000
NUM_REL = 1000
NB = 8192
D = 64
PD = 128
EPW = T // NS
WIN = 128
NWIN = EPW // WIN
RPT = NB // NS
NSLOT = 4

_MESH = plsc.VectorSubcoreMesh(
    core_axis_name="c", subcore_axis_name="s", num_cores=NC, num_subcores=NS
)


def _sc_body(ids_hbm, off_hbm, ents_hbm, etab_hbm, rtab_hbm,
             acc_out, emb_out,
             acc_sh, offs_v, seg_v, ids_v, erows_v, nids_v, gsem, ssem):
    cid = lax.axis_index("c")
    sid = lax.axis_index("s")
    wid = cid * NS + sid
    lo = sid * EPW

    pltpu.sync_copy(off_hbm, offs_v)
    pltpu.sync_copy(ids_hbm.at[cid, sid], ids_v)
    pltpu.sync_copy(ents_hbm.at[cid, sid], nids_v)

    zi = jnp.zeros((L,), jnp.int32)
    zf = jnp.zeros((L,), jnp.float32)
    CPR = WIN // L

    @pl.loop(0, EPW // L)
    def _(i):
        seg_v[i // CPR, pl.ds((i % CPR) * L, L)] = zi

    @pl.loop(0, WIN * D // L)
    def _(i):
        erows_v[0, i // (D // L), pl.ds((i % (D // L)) * L, L)] = zf

    for j in range(RPT // WIN):
        r0 = sid * RPT + j * WIN
        pltpu.sync_copy(erows_v.at[0], acc_sh.at[pl.ds(r0, WIN)])

    ones = jnp.ones((L,), jnp.int32)

    def _scatter_offsets(k, carry):
        v = offs_v[pl.ds(k * L, L)]
        rel = v - lo
        m_in = (rel >= 0) & (rel < EPW)
        plsc.addupdate_scatter(seg_v, [rel // WIN, rel % WIN], ones, mask=m_in)
        return carry + jnp.where(v < lo, 1, 0)

    lt_lanes = lax.fori_loop(0, NB // L, _scatter_offsets,
                             jnp.zeros((L,), jnp.int32))
    c0 = jnp.sum(lt_lanes)

    def _cumsum_row(j, carry):
        v = seg_v[j // CPR, pl.ds((j % CPR) * L, L)]
        c = plsc.cumsum(v) + carry
        seg_v[j // CPR, pl.ds((j % CPR) * L, L)] = c
        return jnp.max(c)

    lax.fori_loop(0, EPW // L, _cumsum_row, c0 - 1)

    plsc.subcore_barrier()

    def _edge_loop(tab_hbm):
        for b in range(NSLOT):
            pltpu.async_copy(tab_hbm.at[ids_v.at[b]], erows_v.at[b],
                             gsem.at[b])

        @pl.loop(0, NWIN // NSLOT)
        def _(g):
            for b in range(NSLOT):
                w = g * NSLOT + b
                src = erows_v.at[b]
                pltpu.make_async_copy(tab_hbm.at[ids_v.at[w]],
                                      erows_v.at[b], gsem.at[b]).wait()
                pltpu.async_copy(src, acc_sh.at[seg_v.at[w]],
                                 ssem.at[b], add=True)
                pltpu.make_async_copy(src, acc_sh.at[seg_v.at[w]],
                                      ssem.at[b]).wait()

                @pl.when(g < NWIN // NSLOT - 1)
                def _():
                    pltpu.async_copy(tab_hbm.at[ids_v.at[w + NSLOT]],
                                     erows_v.at[b], gsem.at[b])

    @pl.when(cid == 0)
    def _():
        _edge_loop(etab_hbm)

    @pl.when(cid == 1)
    def _():
        _edge_loop(rtab_hbm)

    for j in range(NB // NW // WIN):
        pltpu.sync_copy(etab_hbm.at[nids_v.at[j]], erows_v.at[0])
        pltpu.sync_copy(erows_v.at[0],
                        emb_out.at[pl.ds(wid * (NB // NW) + j * WIN, WIN)])

    plsc.subcore_barrier()
    for j in range(RPT // WIN):
        r0 = sid * RPT + j * WIN
        pltpu.sync_copy(acc_sh.at[pl.ds(r0, WIN)], acc_out.at[cid, pl.ds(r0, WIN)])


_sc_call = pl.kernel(
    _sc_body,
    out_type=(
        jax.ShapeDtypeStruct((NC, NB, D), jnp.float32),
        jax.ShapeDtypeStruct((NB, D), jnp.float32),
    ),
    mesh=_MESH,
    compiler_params=pltpu.CompilerParams(
        needs_layout_passes=False, use_tc_tiling_on_sc=False
    ),
    scratch_types=[
        pltpu.VMEM_SHARED((NB, D), jnp.float32),
        pltpu.VMEM((NB,), jnp.int32),
        pltpu.VMEM((NWIN, WIN), jnp.int32),
        pltpu.VMEM((NWIN, WIN), jnp.int32),
        pltpu.VMEM((NSLOT, WIN, D), jnp.float32),
        pltpu.VMEM((NB // NW // WIN, WIN), jnp.int32),
        pltpu.SemaphoreType.DMA((NSLOT,)),
        pltpu.SemaphoreType.DMA((NSLOT,)),
    ],
)


def _tcT_body(xt, eye, out):
    y = lax.dot_general(xt[...].astype(jnp.bfloat16),
                        eye[...].astype(jnp.bfloat16),
                        (((0,), (0,)), ((), ())),
                        preferred_element_type=jnp.float32)
    out[...] = jnp.concatenate([y, jnp.zeros_like(y)], axis=1)


def _pad_rows(table, nrows, bct):
    tT = table.T
    eye = jnp.eye(D, dtype=jnp.float32)
    return pl.pallas_call(
        _tcT_body,
        grid=(pl.cdiv(nrows, bct),),
        in_specs=[
            pl.BlockSpec((D, bct), lambda i: (0, i)),
            pl.BlockSpec((D, D), lambda i: (0, 0)),
        ],
        out_specs=pl.BlockSpec((bct, PD), lambda i: (i, 0)),
        out_shape=jax.ShapeDtypeStruct((nrows, PD), jnp.float32),
    )(tT, eye)


def _tc1_body(acc_e, acc_r, emb, olo, ohi, wt, wne, wnr, bt, bn, node_out):
    cnt = (ohi[...] - olo[...]).astype(jnp.float32)
    inv = 1.0 / jnp.maximum(cnt, 1.0)
    bag_e = acc_e[...] * inv
    bag_r = acc_r[...] * inv
    dn = (((1,), (1,)), ((), ()))
    ent_trans = lax.dot_general(emb[...], wt[...], dn,
                                precision=lax.Precision.HIGHEST,
                                preferred_element_type=jnp.float32)
    neigh = (lax.dot_general(bag_e, wne[...], dn,
                             precision=lax.Precision.HIGHEST,
                             preferred_element_type=jnp.float32)
             + lax.dot_general(bag_r, wnr[...], dn,
                               precision=lax.Precision.HIGHEST,
                               preferred_element_type=jnp.float32))
    node_out[...] = jnp.maximum(ent_trans + neigh + bt[...] + bn[...], 0.0)


def _tc2_body(node2, wr, br, pair_out):
    dn = (((1,), (1,)), ((), ()))
    pair = lax.dot_general(node2[...], wr[...], dn,
                           precision=lax.Precision.HIGHEST,
                           preferred_element_type=jnp.float32)
    pair_out[...] = jnp.maximum(pair + br[...], 0.0)


def kernel(entities, neighbor_entities, neighbor_relations, offsets,
           entity_table, relation_table, W_t, b_t, W_n, b_n, W_r, b_r):
    entities = (entities.astype(jnp.int32) * 2).reshape(
        NC, NS, NB // NW // WIN, WIN)
    ne = neighbor_entities.astype(jnp.int32) * 2
    nr = neighbor_relations.astype(jnp.int32) * 2
    ids = jnp.stack([ne.reshape(NS, NWIN, WIN), nr.reshape(NS, NWIN, WIN)])
    offsets = offsets.astype(jnp.int32)

    etab2 = _pad_rows(entity_table, NUM_ENT, 2048).reshape(2 * NUM_ENT, D)
    rtab2 = _pad_rows(relation_table, NUM_REL, 1000).reshape(2 * NUM_REL, D)

    acc, emb = _sc_call(ids, offsets, entities, etab2, rtab2)
    acc_e = acc[0]
    acc_r = acc[1]

    olo = offsets.reshape(NB, 1)
    ohi = jnp.concatenate([offsets[1:], jnp.full((1,), T, jnp.int32)]).reshape(NB, 1)
    wne = W_n[:, :D]
    wnr = W_n[:, D:]

    BR1 = 1024
    node = pl.pallas_call(
        _tc1_body,
        grid=(NB // BR1,),
        in_specs=[
            pl.BlockSpec((BR1, D), lambda i: (i, 0)),
            pl.BlockSpec((BR1, D), lambda i: (i, 0)),
            pl.BlockSpec((BR1, D), lambda i: (i, 0)),
            pl.BlockSpec((BR1, 1), lambda i: (i, 0)),
            pl.BlockSpec((BR1, 1), lambda i: (i, 0)),
            pl.BlockSpec((2 * D, D), lambda i: (0, 0)),
            pl.BlockSpec((2 * D, D), lambda i: (0, 0)),
            pl.BlockSpec((2 * D, D), lambda i: (0, 0)),
            pl.BlockSpec((1, 2 * D), lambda i: (0, 0)),
            pl.BlockSpec((1, 2 * D), lambda i: (0, 0)),
        ],
        out_specs=pl.BlockSpec((BR1, 2 * D), lambda i: (i, 0)),
        out_shape=jax.ShapeDtypeStruct((NB, 2 * D), jnp.float32),
    )(acc_e, acc_r, emb, olo, ohi, W_t, wne, wnr,
      b_t.reshape(1, 2 * D), b_n.reshape(1, 2 * D))

    node2 = node.reshape(NB // 2, 4 * D)
    BR2 = 1024
    pair = pl.pallas_call(
        _tc2_body,
        grid=(NB // 2 // BR2,),
        in_specs=[
            pl.BlockSpec((BR2, 4 * D), lambda i: (i, 0)),
            pl.BlockSpec((2 * D, 4 * D), lambda i: (0, 0)),
            pl.BlockSpec((1, 2 * D), lambda i: (0, 0)),
        ],
        out_specs=pl.BlockSpec((BR2, 2 * D), lambda i: (i, 0)),
        out_shape=jax.ShapeDtypeStruct((NB // 2, 2 * D), jnp.float32),
    )(node2, W_r, b_r.reshape(1, 2 * D))
    return pair


# speedup vs baseline: 1.9441x; 1.0020x over previous
import jax
import jax.numpy as jnp
from jax import lax
from jax.experimental import pallas as pl
from jax.experimental.pallas import tpu as pltpu
from jax.experimental.pallas import tpu_sc as plsc

NC = 2
NS = 16
L = 16
NW = NC * NS
T = 409600
NUM_ENT = 1000000
NUM_REL = 1000
NB = 8192
D = 64
PD = 128
EPW = T // NS
WIN = 128
NWIN = EPW // WIN
RPT = NB // NS
NSLOT = 4

_MESH = plsc.VectorSubcoreMesh(
    core_axis_name="c", subcore_axis_name="s", num_cores=NC, num_subcores=NS
)


def _sc_body(ids_hbm, off_hbm, ents_hbm, etab_hbm, rtab_hbm,
             acc_out, emb_out,
             acc_sh, offs_v, seg_v, ids_v, erows_v, nids_v, gsem, ssem):
    cid = lax.axis_index("c")
    sid = lax.axis_index("s")
    wid = cid * NS + sid
    lo = sid * EPW

    pltpu.sync_copy(off_hbm, offs_v)
    pltpu.sync_copy(ids_hbm.at[cid, sid], ids_v)
    pltpu.sync_copy(ents_hbm.at[cid, sid], nids_v)

    zi = jnp.zeros((L,), jnp.int32)
    zf = jnp.zeros((L,), jnp.float32)
    CPR = WIN // L

    @pl.loop(0, EPW // L)
    def _(i):
        seg_v[i // CPR, pl.ds((i % CPR) * L, L)] = zi

    @pl.loop(0, WIN * D // L)
    def _(i):
        erows_v[0, i // (D // L), pl.ds((i % (D // L)) * L, L)] = zf

    for j in range(RPT // WIN):
        r0 = sid * RPT + j * WIN
        pltpu.sync_copy(erows_v.at[0], acc_sh.at[pl.ds(r0, WIN)])

    ones = jnp.ones((L,), jnp.int32)

    def _scatter_offsets(k, carry):
        v = offs_v[pl.ds(k * L, L)]
        rel = v - lo
        m_in = (rel >= 0) & (rel < EPW)
        plsc.addupdate_scatter(seg_v, [rel // WIN, rel % WIN], ones, mask=m_in)
        return carry + jnp.where(v < lo, 1, 0)

    lt_lanes = lax.fori_loop(0, NB // L, _scatter_offsets,
                             jnp.zeros((L,), jnp.int32))
    c0 = jnp.sum(lt_lanes)

    def _cumsum_row(j, carry):
        v = seg_v[j // CPR, pl.ds((j % CPR) * L, L)]
        c = plsc.cumsum(v) + carry
        seg_v[j // CPR, pl.ds((j % CPR) * L, L)] = c
        return jnp.max(c)

    lax.fori_loop(0, EPW // L, _cumsum_row, c0 - 1)

    plsc.subcore_barrier()

    def _edge_loop(tab_hbm):
        for b in range(NSLOT):
            pltpu.async_copy(tab_hbm.at[ids_v.at[b]], erows_v.at[b],
                             gsem.at[b])

        @pl.loop(0, NWIN // NSLOT)
        def _(g):
            for b in range(NSLOT):
                w = g * NSLOT + b
                src = erows_v.at[b]
                pltpu.make_async_copy(tab_hbm.at[ids_v.at[w]],
                                      erows_v.at[b], gsem.at[b]).wait()
                pltpu.async_copy(src, acc_sh.at[seg_v.at[w]],
                                 ssem.at[b], add=True)
                pltpu.make_async_copy(src, acc_sh.at[seg_v.at[w]],
                                      ssem.at[b]).wait()

                @pl.when(g < NWIN // NSLOT - 1)
                def _():
                    pltpu.async_copy(tab_hbm.at[ids_v.at[w + NSLOT]],
                                     erows_v.at[b], gsem.at[b])

    @pl.when(cid == 0)
    def _():
        _edge_loop(etab_hbm)

    @pl.when(cid == 1)
    def _():
        _edge_loop(rtab_hbm)

    for j in range(NB // NW // WIN):
        pltpu.sync_copy(etab_hbm.at[nids_v.at[j]], erows_v.at[0])
        pltpu.sync_copy(erows_v.at[0],
                        emb_out.at[pl.ds(wid * (NB // NW) + j * WIN, WIN)])

    plsc.subcore_barrier()
    for j in range(RPT // WIN):
        r0 = sid * RPT + j * WIN
        pltpu.sync_copy(acc_sh.at[pl.ds(r0, WIN)], acc_out.at[cid, pl.ds(r0, WIN)])


_sc_call = pl.kernel(
    _sc_body,
    out_type=(
        jax.ShapeDtypeStruct((NC, NB, D), jnp.float32),
        jax.ShapeDtypeStruct((NB, D), jnp.float32),
    ),
    mesh=_MESH,
    compiler_params=pltpu.CompilerParams(
        needs_layout_passes=False, use_tc_tiling_on_sc=False
    ),
    scratch_types=[
        pltpu.VMEM_SHARED((NB, D), jnp.float32),
        pltpu.VMEM((NB,), jnp.int32),
        pltpu.VMEM((NWIN, WIN), jnp.int32),
        pltpu.VMEM((NWIN, WIN), jnp.int32),
        pltpu.VMEM((NSLOT, WIN, D), jnp.float32),
        pltpu.VMEM((NB // NW // WIN, WIN), jnp.int32),
        pltpu.SemaphoreType.DMA((NSLOT,)),
        pltpu.SemaphoreType.DMA((NSLOT,)),
    ],
)


def _tcT_body(xt, eye, out):
    y = xt[...].T
    out[...] = jnp.concatenate([y, jnp.zeros_like(y)], axis=1)


def _pad_rows(table, nrows, bct):
    tT = table.T
    eye = jnp.eye(D, dtype=jnp.float32)
    return pl.pallas_call(
        _tcT_body,
        grid=(pl.cdiv(nrows, bct),),
        in_specs=[
            pl.BlockSpec((D, bct), lambda i: (0, i)),
            pl.BlockSpec((D, D), lambda i: (0, 0)),
        ],
        out_specs=pl.BlockSpec((bct, PD), lambda i: (i, 0)),
        out_shape=jax.ShapeDtypeStruct((nrows, PD), jnp.float32),
    )(tT, eye)


def _tc1_body(acc_e, acc_r, emb, olo, ohi, wt, wne, wnr, bt, bn, node_out):
    cnt = (ohi[...] - olo[...]).astype(jnp.float32)
    inv = 1.0 / jnp.maximum(cnt, 1.0)
    bag_e = acc_e[...] * inv
    bag_r = acc_r[...] * inv
    dn = (((1,), (1,)), ((), ()))
    ent_trans = lax.dot_general(emb[...], wt[...], dn,
                                precision=lax.Precision.HIGHEST,
                                preferred_element_type=jnp.float32)
    neigh = (lax.dot_general(bag_e, wne[...], dn,
                             precision=lax.Precision.HIGHEST,
                             preferred_element_type=jnp.float32)
             + lax.dot_general(bag_r, wnr[...], dn,
                               precision=lax.Precision.HIGHEST,
                               preferred_element_type=jnp.float32))
    node_out[...] = jnp.maximum(ent_trans + neigh + bt[...] + bn[...], 0.0)


def _tc2_body(node2, wr, br, pair_out):
    dn = (((1,), (1,)), ((), ()))
    pair = lax.dot_general(node2[...], wr[...], dn,
                           precision=lax.Precision.HIGHEST,
                           preferred_element_type=jnp.float32)
    pair_out[...] = jnp.maximum(pair + br[...], 0.0)


def kernel(entities, neighbor_entities, neighbor_relations, offsets,
           entity_table, relation_table, W_t, b_t, W_n, b_n, W_r, b_r):
    entities = (entities.astype(jnp.int32) * 2).reshape(
        NC, NS, NB // NW // WIN, WIN)
    ne = neighbor_entities.astype(jnp.int32) * 2
    nr = neighbor_relations.astype(jnp.int32) * 2
    ids = jnp.stack([ne.reshape(NS, NWIN, WIN), nr.reshape(NS, NWIN, WIN)])
    offsets = offsets.astype(jnp.int32)

    etab2 = _pad_rows(entity_table, NUM_ENT, 2048).reshape(2 * NUM_ENT, D)
    rtab2 = _pad_rows(relation_table, NUM_REL, 1000).reshape(2 * NUM_REL, D)

    acc, emb = _sc_call(ids, offsets, entities, etab2, rtab2)
    acc_e = acc[0]
    acc_r = acc[1]

    olo = offsets.reshape(NB, 1)
    ohi = jnp.concatenate([offsets[1:], jnp.full((1,), T, jnp.int32)]).reshape(NB, 1)
    wne = W_n[:, :D]
    wnr = W_n[:, D:]

    BR1 = 1024
    node = pl.pallas_call(
        _tc1_body,
        grid=(NB // BR1,),
        in_specs=[
            pl.BlockSpec((BR1, D), lambda i: (i, 0)),
            pl.BlockSpec((BR1, D), lambda i: (i, 0)),
            pl.BlockSpec((BR1, D), lambda i: (i, 0)),
            pl.BlockSpec((BR1, 1), lambda i: (i, 0)),
            pl.BlockSpec((BR1, 1), lambda i: (i, 0)),
            pl.BlockSpec((2 * D, D), lambda i: (0, 0)),
            pl.BlockSpec((2 * D, D), lambda i: (0, 0)),
            pl.BlockSpec((2 * D, D), lambda i: (0, 0)),
            pl.BlockSpec((1, 2 * D), lambda i: (0, 0)),
            pl.BlockSpec((1, 2 * D), lambda i: (0, 0)),
        ],
        out_specs=pl.BlockSpec((BR1, 2 * D), lambda i: (i, 0)),
        out_shape=jax.ShapeDtypeStruct((NB, 2 * D), jnp.float32),
    )(acc_e, acc_r, emb, olo, ohi, W_t, wne, wnr,
      b_t.reshape(1, 2 * D), b_n.reshape(1, 2 * D))

    node2 = node.reshape(NB // 2, 4 * D)
    BR2 = 1024
    pair = pl.pallas_call(
        _tc2_body,
        grid=(NB // 2 // BR2,),
        in_specs=[
            pl.BlockSpec((BR2, 4 * D), lambda i: (i, 0)),
            pl.BlockSpec((2 * D, 4 * D), lambda i: (0, 0)),
            pl.BlockSpec((1, 2 * D), lambda i: (0, 0)),
        ],
        out_specs=pl.BlockSpec((BR2, 2 * D), lambda i: (i, 0)),
        out_shape=jax.ShapeDtypeStruct((NB // 2, 2 * D), jnp.float32),
    )(node2, W_r, b_r.reshape(1, 2 * D))
    return pair


# speedup vs baseline: 2.6996x; 1.3886x over previous
import jax
import jax.numpy as jnp
from jax import lax
from jax.experimental import pallas as pl
from jax.experimental.pallas import tpu as pltpu
from jax.experimental.pallas import tpu_sc as plsc

NC = 2
NS = 16
L = 16
NW = NC * NS
T = 409600
NUM_ENT = 1000000
NUM_REL = 1000
NB = 8192
D = 64
PD = 128
EPW = T // NS
WIN = 128
NWIN = EPW // WIN
RPT = NB // NS
NSLOT = 4

_MESH = plsc.VectorSubcoreMesh(
    core_axis_name="c", subcore_axis_name="s", num_cores=NC, num_subcores=NS
)


def _sc_body(ids_hbm, off_hbm, ents_hbm, etab_hbm, rtab_hbm,
             acc_out, emb_out,
             acc_sh, offs_v, seg_v, ids_v, erows_v, nids_v, gsem, ssem):
    cid = lax.axis_index("c")
    sid = lax.axis_index("s")
    wid = cid * NS + sid
    lo = sid * EPW

    pltpu.sync_copy(off_hbm, offs_v)
    pltpu.sync_copy(ids_hbm.at[cid, sid], ids_v)
    pltpu.sync_copy(ents_hbm.at[cid, sid], nids_v)

    zi = jnp.zeros((L,), jnp.int32)
    zf = jnp.zeros((L,), jnp.float32)
    CPR = WIN // L

    @pl.loop(0, EPW // L)
    def _(i):
        seg_v[i // CPR, pl.ds((i % CPR) * L, L)] = zi

    @pl.loop(0, WIN * D // L)
    def _(i):
        erows_v[0, i // (D // L), pl.ds((i % (D // L)) * L, L)] = zf

    for j in range(RPT // WIN):
        r0 = sid * RPT + j * WIN
        pltpu.sync_copy(erows_v.at[0], acc_sh.at[pl.ds(r0, WIN)])

    ones = jnp.ones((L,), jnp.int32)

    def _scatter_offsets(k, carry):
        v = offs_v[pl.ds(k * L, L)]
        rel = v - lo
        m_in = (rel >= 0) & (rel < EPW)
        plsc.addupdate_scatter(seg_v, [rel // WIN, rel % WIN], ones, mask=m_in)
        return carry + jnp.where(v < lo, 1, 0)

    lt_lanes = lax.fori_loop(0, NB // L, _scatter_offsets,
                             jnp.zeros((L,), jnp.int32))
    c0 = jnp.sum(lt_lanes)

    def _cumsum_row(j, carry):
        v = seg_v[j // CPR, pl.ds((j % CPR) * L, L)]
        c = plsc.cumsum(v) + carry
        seg_v[j // CPR, pl.ds((j % CPR) * L, L)] = c
        return jnp.max(c)

    lax.fori_loop(0, EPW // L, _cumsum_row, c0 - 1)

    plsc.subcore_barrier()

    def _edge_loop(tab_hbm):
        for b in range(NSLOT):
            pltpu.async_copy(tab_hbm.at[ids_v.at[b]], erows_v.at[b],
                             gsem.at[b])

        @pl.loop(0, NWIN // NSLOT)
        def _(g):
            for b in range(NSLOT):
                w = g * NSLOT + b
                src = erows_v.at[b]
                pltpu.make_async_copy(tab_hbm.at[ids_v.at[w]],
                                      erows_v.at[b], gsem.at[b]).wait()
                pltpu.async_copy(src, acc_sh.at[seg_v.at[w]],
                                 ssem.at[b], add=True)
                pltpu.make_async_copy(src, acc_sh.at[seg_v.at[w]],
                                      ssem.at[b]).wait()

                @pl.when(g < NWIN // NSLOT - 1)
                def _():
                    pltpu.async_copy(tab_hbm.at[ids_v.at[w + NSLOT]],
                                     erows_v.at[b], gsem.at[b])

    @pl.when(cid == 0)
    def _():
        _edge_loop(etab_hbm)

    @pl.when(cid == 1)
    def _():
        _edge_loop(rtab_hbm)

    for j in range(NB // NW // WIN):
        pltpu.sync_copy(etab_hbm.at[nids_v.at[j]], erows_v.at[0])
        pltpu.sync_copy(erows_v.at[0],
                        emb_out.at[pl.ds(wid * (NB // NW) + j * WIN, WIN)])

    plsc.subcore_barrier()
    for j in range(RPT // WIN):
        r0 = sid * RPT + j * WIN
        pltpu.sync_copy(acc_sh.at[pl.ds(r0, WIN)], acc_out.at[cid, pl.ds(r0, WIN)])


_sc_call = pl.kernel(
    _sc_body,
    out_type=(
        jax.ShapeDtypeStruct((NC, NB, D), jnp.float32),
        jax.ShapeDtypeStruct((NB, D), jnp.float32),
    ),
    mesh=_MESH,
    compiler_params=pltpu.CompilerParams(
        needs_layout_passes=False, use_tc_tiling_on_sc=False
    ),
    scratch_types=[
        pltpu.VMEM_SHARED((NB, D), jnp.float32),
        pltpu.VMEM((NB,), jnp.int32),
        pltpu.VMEM((NWIN, WIN), jnp.int32),
        pltpu.VMEM((NWIN, WIN), jnp.int32),
        pltpu.VMEM((NSLOT, WIN, D), jnp.float32),
        pltpu.VMEM((NB // NW // WIN, WIN), jnp.int32),
        pltpu.SemaphoreType.DMA((NSLOT,)),
        pltpu.SemaphoreType.DMA((NSLOT,)),
    ],
)


def _tcT_body(xt, eye, out):
    y = xt[...].T
    out[...] = jnp.concatenate([y, jnp.zeros_like(y)], axis=1)


def _pad_rows(table, nrows, bct):
    tT = table.T
    eye = jnp.eye(D, dtype=jnp.float32)
    return pl.pallas_call(
        _tcT_body,
        grid=(pl.cdiv(nrows, bct),),
        in_specs=[
            pl.BlockSpec((D, bct), lambda i: (0, i)),
            pl.BlockSpec((D, D), lambda i: (0, 0)),
        ],
        out_specs=pl.BlockSpec((bct, PD), lambda i: (i, 0)),
        out_shape=jax.ShapeDtypeStruct((nrows, PD), jnp.float32),
    )(tT, eye)


def _tc1_body(acc_e, acc_r, emb, olo, ohi, wt, wne, wnr, bt, bn, node_out):
    cnt = (ohi[...] - olo[...]).astype(jnp.float32)
    inv = 1.0 / jnp.maximum(cnt, 1.0)
    bag_e = acc_e[...] * inv
    bag_r = acc_r[...] * inv
    dn = (((1,), (1,)), ((), ()))
    ent_trans = lax.dot_general(emb[...], wt[...], dn,
                                precision=lax.Precision.HIGHEST,
                                preferred_element_type=jnp.float32)
    neigh = (lax.dot_general(bag_e, wne[...], dn,
                             precision=lax.Precision.HIGHEST,
                             preferred_element_type=jnp.float32)
             + lax.dot_general(bag_r, wnr[...], dn,
                               precision=lax.Precision.HIGHEST,
                               preferred_element_type=jnp.float32))
    node_out[...] = jnp.maximum(ent_trans + neigh + bt[...] + bn[...], 0.0)


def _tc2_body(node2, wr, br, pair_out):
    dn = (((1,), (1,)), ((), ()))
    pair = lax.dot_general(node2[...], wr[...], dn,
                           precision=lax.Precision.HIGHEST,
                           preferred_element_type=jnp.float32)
    pair_out[...] = jnp.maximum(pair + br[...], 0.0)


def kernel(entities, neighbor_entities, neighbor_relations, offsets,
           entity_table, relation_table, W_t, b_t, W_n, b_n, W_r, b_r):
    entities = (entities.astype(jnp.int32) * 2).reshape(
        NC, NS, NB // NW // WIN, WIN)
    ne = neighbor_entities.astype(jnp.int32) * 2
    nr = neighbor_relations.astype(jnp.int32) * 2
    ids = jnp.stack([ne.reshape(NS, NWIN, WIN), nr.reshape(NS, NWIN, WIN)])
    offsets = offsets.astype(jnp.int32)

    etab2 = _pad_rows(entity_table, NUM_ENT, 8192).reshape(2 * NUM_ENT, D)
    rtab2 = _pad_rows(relation_table, NUM_REL, 1000).reshape(2 * NUM_REL, D)

    acc, emb = _sc_call(ids, offsets, entities, etab2, rtab2)
    acc_e = acc[0]
    acc_r = acc[1]

    olo = offsets.reshape(NB, 1)
    ohi = jnp.concatenate([offsets[1:], jnp.full((1,), T, jnp.int32)]).reshape(NB, 1)
    wne = W_n[:, :D]
    wnr = W_n[:, D:]

    BR1 = 1024
    node = pl.pallas_call(
        _tc1_body,
        grid=(NB // BR1,),
        in_specs=[
            pl.BlockSpec((BR1, D), lambda i: (i, 0)),
            pl.BlockSpec((BR1, D), lambda i: (i, 0)),
            pl.BlockSpec((BR1, D), lambda i: (i, 0)),
            pl.BlockSpec((BR1, 1), lambda i: (i, 0)),
            pl.BlockSpec((BR1, 1), lambda i: (i, 0)),
            pl.BlockSpec((2 * D, D), lambda i: (0, 0)),
            pl.BlockSpec((2 * D, D), lambda i: (0, 0)),
            pl.BlockSpec((2 * D, D), lambda i: (0, 0)),
            pl.BlockSpec((1, 2 * D), lambda i: (0, 0)),
            pl.BlockSpec((1, 2 * D), lambda i: (0, 0)),
        ],
        out_specs=pl.BlockSpec((BR1, 2 * D), lambda i: (i, 0)),
        out_shape=jax.ShapeDtypeStruct((NB, 2 * D), jnp.float32),
    )(acc_e, acc_r, emb, olo, ohi, W_t, wne, wnr,
      b_t.reshape(1, 2 * D), b_n.reshape(1, 2 * D))

    node2 = node.reshape(NB // 2, 4 * D)
    BR2 = 1024
    pair = pl.pallas_call(
        _tc2_body,
        grid=(NB // 2 // BR2,),
        in_specs=[
            pl.BlockSpec((BR2, 4 * D), lambda i: (i, 0)),
            pl.BlockSpec((2 * D, 4 * D), lambda i: (0, 0)),
            pl.BlockSpec((1, 2 * D), lambda i: (0, 0)),
        ],
        out_specs=pl.BlockSpec((BR2, 2 * D), lambda i: (i, 0)),
        out_shape=jax.ShapeDtypeStruct((NB // 2, 2 * D), jnp.float32),
    )(node2, W_r, b_r.reshape(1, 2 * D))
    return pair
